# Initial kernel scaffold; baseline (speedup 1.0000x reference)
import jax
import jax.numpy as jnp
from jax.experimental import pallas as pl


def kernel(state, state_prev, wfb, w1, w2, bias):
    raise NotImplementedError("write your pallas kernel here")


# speedup vs baseline: 1.5791x; 1.5791x over previous
import jax
import jax.numpy as jnp
from jax.experimental import pallas as pl
from jax.experimental.pallas import tpu as pltpu

_OUTPUTS = 4


def _a2c_fused(s_ref, sp_ref, wfb_ref, w1_ref, w2_ref, bias_ref,
               pol_ref, crit_ref, im_ref):
    x = jnp.concatenate([s_ref[...], sp_ref[...]], axis=1)
    ff = jnp.maximum(
        jnp.dot(x, wfb_ref[...], preferred_element_type=jnp.float32)
        + bias_ref[:, 0:128],
        0.0)

    n_h1 = w1_ref.shape[1]
    h = jnp.maximum(
        jnp.dot(ff, w1_ref[...], preferred_element_type=jnp.float32)
        + bias_ref[:, 128:128 + n_h1],
        0.0)

    out = (jnp.dot(h, w2_ref[...], preferred_element_type=jnp.float32)
           + bias_ref[:, 256:])
    no = _OUTPUTS
    pol_ref[...] = out[:, 0:no]
    crit_ref[...] = out[:, no:no + 1]
    im_ref[...] = out[:, no + 1:2 * no + 1]


def kernel(state, state_prev, wfb, w1, w2, bias):
    B, D = state.shape
    no = _OUTPUTS

    tb = B
    for cand in (2048, 1024, 512, 256, 128, 64, 32, 16, 8):
        if B % cand == 0:
            tb = cand
            break

    out_shapes = [
        jax.ShapeDtypeStruct((B, no), jnp.float32),
        jax.ShapeDtypeStruct((B, 1), jnp.float32),
        jax.ShapeDtypeStruct((B, no), jnp.float32),
    ]
    outs = pl.pallas_call(
        _a2c_fused,
        out_shape=out_shapes,
        grid=(B // tb,),
        in_specs=[
            pl.BlockSpec((tb, D), lambda i: (i, 0)),
            pl.BlockSpec((tb, D), lambda i: (i, 0)),
            pl.BlockSpec(wfb.shape, lambda i: (0, 0)),
            pl.BlockSpec(w1.shape, lambda i: (0, 0)),
            pl.BlockSpec(w2.shape, lambda i: (0, 0)),
            pl.BlockSpec(bias.shape, lambda i: (0, 0)),
        ],
        out_specs=[
            pl.BlockSpec((tb, no), lambda i: (i, 0)),
            pl.BlockSpec((tb, 1), lambda i: (i, 0)),
            pl.BlockSpec((tb, no), lambda i: (i, 0)),
        ],
        compiler_params=pltpu.CompilerParams(
            dimension_semantics=("parallel",)),
    )(state, state_prev, wfb, w1, w2, bias)
    return outs[0], outs[1], outs[2]


# speedup vs baseline: 1.7066x; 1.0807x over previous
import jax
import jax.numpy as jnp
from jax.experimental import pallas as pl
from jax.experimental.pallas import tpu as pltpu

_OUTPUTS = 4


def _a2c_fused(s_ref, sp_ref, wfb_ref, w1_ref, w2_ref, bias_ref,
               pol_ref, crit_ref, im_ref):
    x = jnp.concatenate([s_ref[...], sp_ref[...]], axis=1)
    ff = jnp.maximum(
        jnp.dot(x, wfb_ref[...], preferred_element_type=jnp.float32)
        + bias_ref[:, 0:128],
        0.0)

    n_h1 = w1_ref.shape[1]
    h = jnp.maximum(
        jnp.dot(ff, w1_ref[...], preferred_element_type=jnp.float32)
        + bias_ref[:, 128:128 + n_h1],
        0.0)

    out = (jnp.dot(h, w2_ref[...], preferred_element_type=jnp.float32)
           + bias_ref[:, 256:])
    no = _OUTPUTS
    pol_ref[...] = out[:, 0:no]
    crit_ref[...] = out[:, no:no + 1]
    im_ref[...] = out[:, no + 1:2 * no + 1]


def kernel(state, state_prev, wfb, w1, w2, bias):
    B, D = state.shape
    no = _OUTPUTS

    tb = B
    for cand in (4096, 2048, 1024, 512, 256, 128, 64, 32, 16, 8):
        if B % cand == 0:
            tb = cand
            break

    out_shapes = [
        jax.ShapeDtypeStruct((B, no), jnp.float32),
        jax.ShapeDtypeStruct((B, 1), jnp.float32),
        jax.ShapeDtypeStruct((B, no), jnp.float32),
    ]
    outs = pl.pallas_call(
        _a2c_fused,
        out_shape=out_shapes,
        grid=(B // tb,),
        in_specs=[
            pl.BlockSpec((tb, D), lambda i: (i, 0)),
            pl.BlockSpec((tb, D), lambda i: (i, 0)),
            pl.BlockSpec(wfb.shape, lambda i: (0, 0)),
            pl.BlockSpec(w1.shape, lambda i: (0, 0)),
            pl.BlockSpec(w2.shape, lambda i: (0, 0)),
            pl.BlockSpec(bias.shape, lambda i: (0, 0)),
        ],
        out_specs=[
            pl.BlockSpec((tb, no), lambda i: (i, 0)),
            pl.BlockSpec((tb, 1), lambda i: (i, 0)),
            pl.BlockSpec((tb, no), lambda i: (i, 0)),
        ],
        compiler_params=pltpu.CompilerParams(
            dimension_semantics=("parallel",)),
    )(state, state_prev, wfb, w1, w2, bias)
    return outs[0], outs[1], outs[2]


# speedup vs baseline: 1.7234x; 1.0098x over previous
import jax
import jax.numpy as jnp
from jax.experimental import pallas as pl
from jax.experimental.pallas import tpu as pltpu

_OUTPUTS = 4


def _a2c_fused(s_ref, sp_ref, wfb_ref, w1_ref, w2_ref, bias_ref,
               pol_ref, crit_ref, im_ref):
    x = jnp.concatenate([s_ref[...], sp_ref[...]], axis=1)
    ff = jnp.maximum(
        jnp.dot(x, wfb_ref[...], preferred_element_type=jnp.float32)
        + bias_ref[:, 0:128],
        0.0)

    n_h1 = w1_ref.shape[1]
    h = jnp.maximum(
        jnp.dot(ff, w1_ref[...], preferred_element_type=jnp.float32)
        + bias_ref[:, 128:128 + n_h1],
        0.0)

    out = (jnp.dot(h, w2_ref[...], preferred_element_type=jnp.float32)
           + bias_ref[:, 256:])
    no = _OUTPUTS
    pol_ref[...] = out[:, 0:no]
    crit_ref[...] = out[:, no:no + 1]
    im_ref[...] = out[:, no + 1:2 * no + 1]


def kernel(state, state_prev, wfb, w1, w2, bias):
    B, D = state.shape
    no = _OUTPUTS

    tb = B
    for cand in (8192, 4096, 2048, 1024, 512, 256, 128, 64, 32, 16, 8):
        if B % cand == 0:
            tb = cand
            break

    out_shapes = [
        jax.ShapeDtypeStruct((B, no), jnp.float32),
        jax.ShapeDtypeStruct((B, 1), jnp.float32),
        jax.ShapeDtypeStruct((B, no), jnp.float32),
    ]
    outs = pl.pallas_call(
        _a2c_fused,
        out_shape=out_shapes,
        grid=(B // tb,),
        in_specs=[
            pl.BlockSpec((tb, D), lambda i: (i, 0)),
            pl.BlockSpec((tb, D), lambda i: (i, 0)),
            pl.BlockSpec(wfb.shape, lambda i: (0, 0)),
            pl.BlockSpec(w1.shape, lambda i: (0, 0)),
            pl.BlockSpec(w2.shape, lambda i: (0, 0)),
            pl.BlockSpec(bias.shape, lambda i: (0, 0)),
        ],
        out_specs=[
            pl.BlockSpec((tb, no), lambda i: (i, 0)),
            pl.BlockSpec((tb, 1), lambda i: (i, 0)),
            pl.BlockSpec((tb, no), lambda i: (i, 0)),
        ],
        compiler_params=pltpu.CompilerParams(
            dimension_semantics=("parallel",)),
    )(state, state_prev, wfb, w1, w2, bias)
    return outs[0], outs[1], outs[2]
